# baseline (speedup 1.0000x reference)
import functools

import jax
import jax.numpy as jnp
from jax import lax
from jax.experimental import pallas as pl
from jax.experimental.pallas import tpu as pltpu
from jax.experimental.pallas import tpu_sc as plsc

B = 16384
D = 32
NC = 2
NS = 16
NW = NC * NS
BPW = B // NW
L = 16
IDX_CHUNK = 128
NCHUNK = BPW // IDX_CHUNK

_mesh = plsc.VectorSubcoreMesh(core_axis_name="c", subcore_axis_name="s")


@functools.partial(
    pl.kernel,
    mesh=_mesh,
    out_type=jax.ShapeDtypeStruct((B,), jnp.float32),
    compiler_params=pltpu.CompilerParams(
        needs_layout_passes=False, use_tc_tiling_on_sc=False
    ),
    scratch_types=[
        pltpu.VMEM((BPW, 2), jnp.int32),
        pltpu.VMEM((BPW,), jnp.int32),
        pltpu.VMEM((BPW,), jnp.int32),
        pltpu.VMEM((BPW, D), jnp.float32),
        pltpu.VMEM((BPW, D), jnp.float32),
        pltpu.VMEM((BPW,), jnp.float32),
        pltpu.SemaphoreType.DMA,
        pltpu.SemaphoreType.DMA,
    ],
)
def _sc_dot_kernel(pairs_hbm, t_tab_hbm, h_tab_hbm, out_hbm,
                   pairs_v, t_idx_v, h_idx_v, t_rows, h_rows, out_v,
                   sem_t, sem_h):
    wid = lax.axis_index("s") * NC + lax.axis_index("c")
    base = wid * BPW

    pltpu.sync_copy(pairs_hbm.at[wid], pairs_v)

    lane = lax.iota(jnp.int32, L)
    col0 = jnp.zeros((L,), jnp.int32)
    col1 = jnp.ones((L,), jnp.int32)

    for j in range(NCHUNK):
        def deint_body(k, carry):
            rid = j * IDX_CHUNK + k * L + lane
            t_idx_v[pl.ds(j * IDX_CHUNK + k * L, L)] = plsc.load_gather(
                pairs_v, [rid, col0])
            h_idx_v[pl.ds(j * IDX_CHUNK + k * L, L)] = plsc.load_gather(
                pairs_v, [rid, col1])
            return carry

        lax.fori_loop(0, IDX_CHUNK // L, deint_body, 0)
        pltpu.async_copy(
            t_tab_hbm.at[t_idx_v.at[pl.ds(j * IDX_CHUNK, IDX_CHUNK)]],
            t_rows.at[pl.ds(j * IDX_CHUNK, IDX_CHUNK)],
            sem_t,
        )
        pltpu.async_copy(
            h_tab_hbm.at[h_idx_v.at[pl.ds(j * IDX_CHUNK, IDX_CHUNK)]],
            h_rows.at[pl.ds(j * IDX_CHUNK, IDX_CHUNK)],
            sem_h,
        )

    def chunk_body(c, carry):
        row_ids = c * L + lane
        acc = jnp.zeros((L,), jnp.float32)
        for d in range(D):
            col = jnp.full((L,), d, jnp.int32)
            tv = plsc.load_gather(t_rows, [row_ids, col])
            hv = plsc.load_gather(h_rows, [row_ids, col])
            acc = acc + tv * hv
        out_v[pl.ds(c * L, L)] = acc
        return carry

    for j in range(NCHUNK):
        pltpu.make_async_copy(
            t_tab_hbm.at[t_idx_v.at[pl.ds(j * IDX_CHUNK, IDX_CHUNK)]],
            t_rows.at[pl.ds(j * IDX_CHUNK, IDX_CHUNK)],
            sem_t,
        ).wait()
        pltpu.make_async_copy(
            h_tab_hbm.at[h_idx_v.at[pl.ds(j * IDX_CHUNK, IDX_CHUNK)]],
            h_rows.at[pl.ds(j * IDX_CHUNK, IDX_CHUNK)],
            sem_h,
        ).wait()
        nrow = IDX_CHUNK // L
        lax.fori_loop(j * nrow, (j + 1) * nrow, chunk_body, 0)

    pltpu.sync_copy(out_v, out_hbm.at[pl.ds(base, BPW)])


def kernel(inputs, tumor_table, hospital_table):
    pairs = inputs.reshape(NW, BPW, 2)
    out = _sc_dot_kernel(pairs, tumor_table, hospital_table)
    return out[:, None]


# speedup vs baseline: 1.0442x; 1.0442x over previous
import functools

import jax
import jax.numpy as jnp
from jax import lax
from jax.experimental import pallas as pl
from jax.experimental.pallas import tpu as pltpu
from jax.experimental.pallas import tpu_sc as plsc

B = 16384
D = 32
NC = 2
NS = 16
NW = NC * NS
BPW = B // NW
L = 16
PACK = 4
VROWS = 100000 // PACK
IDX_CHUNK = 64
NCHUNK = BPW // IDX_CHUNK

_mesh = plsc.VectorSubcoreMesh(core_axis_name="c", subcore_axis_name="s")


@functools.partial(
    pl.kernel,
    mesh=_mesh,
    out_type=jax.ShapeDtypeStruct((B,), jnp.float32),
    compiler_params=pltpu.CompilerParams(
        needs_layout_passes=False, use_tc_tiling_on_sc=True
    ),
    scratch_types=[
        pltpu.VMEM((BPW, 2), jnp.int32),
        pltpu.VMEM((BPW,), jnp.int32),
        pltpu.VMEM((BPW,), jnp.int32),
        pltpu.VMEM((BPW,), jnp.int32),
        pltpu.VMEM((BPW,), jnp.int32),
        pltpu.VMEM((IDX_CHUNK, PACK * D), jnp.float32),
        pltpu.VMEM((IDX_CHUNK, PACK * D), jnp.float32),
        pltpu.VMEM((IDX_CHUNK, PACK * D), jnp.float32),
        pltpu.VMEM((IDX_CHUNK, PACK * D), jnp.float32),
        pltpu.VMEM((BPW,), jnp.float32),
        pltpu.SemaphoreType.DMA,
        pltpu.SemaphoreType.DMA,
    ],
)
def _sc_dot_kernel(pairs_hbm, t_tab_hbm, h_tab_hbm, out_hbm,
                   pairs_v, t_idx_v, h_idx_v, t_off_v, h_off_v,
                   t_rows_a, t_rows_b, h_rows_a, h_rows_b, out_v,
                   sem_t, sem_h):
    wid = lax.axis_index("s") * NC + lax.axis_index("c")
    base = wid * BPW

    pltpu.sync_copy(pairs_hbm.at[wid], pairs_v)

    lane = lax.iota(jnp.int32, L)
    col0 = jnp.zeros((L,), jnp.int32)
    col1 = jnp.ones((L,), jnp.int32)

    def deint_body(k, carry):
        rid = k * L + lane
        ti = plsc.load_gather(pairs_v, [rid, col0])
        hi = plsc.load_gather(pairs_v, [rid, col1])
        t_idx_v[pl.ds(k * L, L)] = ti >> 2
        h_idx_v[pl.ds(k * L, L)] = hi >> 2
        t_off_v[pl.ds(k * L, L)] = (ti & 3) << 5
        h_off_v[pl.ds(k * L, L)] = (hi & 3) << 5
        return carry

    lax.fori_loop(0, BPW // L, deint_body, 0)

    t_bufs = (t_rows_a, t_rows_b)
    h_bufs = (h_rows_a, h_rows_b)

    def fire(j):
        pltpu.async_copy(
            t_tab_hbm.at[t_idx_v.at[pl.ds(j * IDX_CHUNK, IDX_CHUNK)]],
            t_bufs[j % 2],
            sem_t,
        )
        pltpu.async_copy(
            h_tab_hbm.at[h_idx_v.at[pl.ds(j * IDX_CHUNK, IDX_CHUNK)]],
            h_bufs[j % 2],
            sem_h,
        )

    def drain(j):
        pltpu.make_async_copy(
            t_tab_hbm.at[t_idx_v.at[pl.ds(j * IDX_CHUNK, IDX_CHUNK)]],
            t_bufs[j % 2],
            sem_t,
        ).wait()
        pltpu.make_async_copy(
            h_tab_hbm.at[h_idx_v.at[pl.ds(j * IDX_CHUNK, IDX_CHUNK)]],
            h_bufs[j % 2],
            sem_h,
        ).wait()

    fire(0)
    fire(1)

    for j in range(NCHUNK):
        drain(j)
        tb = t_bufs[j % 2]
        hb = h_bufs[j % 2]

        def chunk_body(g, carry, _j=j, _tb=tb, _hb=hb):
            rid = g * L + lane
            t_off = t_off_v[pl.ds(_j * IDX_CHUNK + g * L, L)]
            h_off = h_off_v[pl.ds(_j * IDX_CHUNK + g * L, L)]
            acc = jnp.zeros((L,), jnp.float32)
            for d in range(D):
                tv = plsc.load_gather(_tb, [rid, t_off + d])
                hv = plsc.load_gather(_hb, [rid, h_off + d])
                acc = acc + tv * hv
            out_v[pl.ds(_j * IDX_CHUNK + g * L, L)] = acc
            return carry

        lax.fori_loop(0, IDX_CHUNK // L, chunk_body, 0)
        if j + 2 < NCHUNK:
            fire(j + 2)

    pltpu.sync_copy(out_v, out_hbm.at[pl.ds(base, BPW)])


def kernel(inputs, tumor_table, hospital_table):
    pairs = inputs.reshape(NW, BPW, 2)
    t_view = tumor_table.reshape(VROWS, PACK * D)
    h_view = hospital_table.reshape(VROWS, PACK * D)
    out = _sc_dot_kernel(pairs, t_view, h_view)
    return out[:, None]
